# baseline (speedup 1.0000x reference)
import functools

import jax
import jax.numpy as jnp
from jax import lax
from jax.experimental import pallas as pl
from jax.experimental.pallas import tpu as pltpu

N_OUT_ = 128
N_MID_STEPS_ = 6
BN_ = 256

_CP = pltpu.CompilerParams(vmem_limit_bytes=110 * 1024 * 1024)


def _edge_body(x_ref, wt_ref, o_ref, *, sigmoid, out_dtype):
    acc = jnp.dot(x_ref[...], wt_ref[...], preferred_element_type=jnp.float32)
    if sigmoid:
        acc = jax.nn.sigmoid(acc)
    o_ref[...] = acc.astype(out_dtype)


def _edge_step(x, Wt, *, k_size, n_block_start, n_blocks, bn, out_dtype):
    B = x.shape[0]
    body = functools.partial(_edge_body, sigmoid=True, out_dtype=out_dtype)
    return pl.pallas_call(
        body,
        grid=(n_blocks,),
        in_specs=[
            pl.BlockSpec((B, k_size), lambda n: (0, 0)),
            pl.BlockSpec((k_size, bn), lambda n, _s=n_block_start: (0, n + _s)),
        ],
        out_specs=pl.BlockSpec((B, bn), lambda n: (0, n)),
        out_shape=jax.ShapeDtypeStruct((B, n_blocks * bn), out_dtype),
        compiler_params=_CP,
    )(x, Wt)


def _mid_body(h1_ref, wt_ref, o_ref, h_scr, *, n_blocks, bn, n_steps):
    s = pl.program_id(0)
    n = pl.program_id(1)

    @pl.when(jnp.logical_and(s == 0, n == 0))
    def _():
        h_scr[0] = h1_ref[...]

    src = lax.rem(s, 2)
    dst = 1 - src
    acc = jnp.dot(
        h_scr[src],
        wt_ref[:, pl.ds(n * bn, bn)],
        preferred_element_type=jnp.float32,
    )
    tile = jax.nn.sigmoid(acc).astype(jnp.bfloat16)
    h_scr[dst, :, pl.ds(n * bn, bn)] = tile
    o_ref[...] = tile


def _mid_steps(h1, Wt):
    B, H = h1.shape
    n_blocks = H // BN_
    body = functools.partial(
        _mid_body, n_blocks=n_blocks, bn=BN_, n_steps=N_MID_STEPS_)
    return pl.pallas_call(
        body,
        grid=(N_MID_STEPS_, n_blocks),
        in_specs=[
            pl.BlockSpec((B, H), lambda s, n: (0, 0)),
            pl.BlockSpec((H, H), lambda s, n: (0, 0)),
        ],
        out_specs=pl.BlockSpec((B, BN_), lambda s, n: (0, n)),
        out_shape=jax.ShapeDtypeStruct((B, H), jnp.bfloat16),
        scratch_shapes=[pltpu.VMEM((2, B, H), jnp.bfloat16)],
        compiler_params=pltpu.CompilerParams(
            dimension_semantics=("arbitrary", "arbitrary"),
            vmem_limit_bytes=110 * 1024 * 1024,
        ),
    )(h1, Wt)


def kernel(inp, W):
    B, n_inputs = inp.shape
    H = W.shape[0]
    Wt = W.T.astype(jnp.bfloat16)
    inp_bf = inp.astype(jnp.bfloat16)
    h1 = _edge_step(inp_bf, Wt[:n_inputs, :], k_size=n_inputs,
                    n_block_start=0, n_blocks=H // BN_, bn=BN_,
                    out_dtype=jnp.bfloat16)
    h7 = _mid_steps(h1, Wt)
    out = _edge_step(h7, Wt, k_size=H,
                     n_block_start=(H - N_OUT_) // N_OUT_, n_blocks=1,
                     bn=N_OUT_, out_dtype=jnp.float32)
    return out


# speedup vs baseline: 1.0524x; 1.0524x over previous
import functools

import jax
import jax.numpy as jnp
from jax import lax
from jax.experimental import pallas as pl
from jax.experimental.pallas import tpu as pltpu

N_OUT_ = 128
N_MID_STEPS_ = 6
BN_ = 256

_CP = pltpu.CompilerParams(vmem_limit_bytes=110 * 1024 * 1024)
_NT = (((1,), (1,)), ((), ()))


def _edge_body(x_ref, w_ref, o_ref, *, out_dtype):
    acc = lax.dot_general(x_ref[...], w_ref[...], dimension_numbers=_NT,
                          preferred_element_type=jnp.float32)
    o_ref[...] = jax.nn.sigmoid(acc).astype(out_dtype)


def _edge_step(x, W, *, k_size, n_block_start, n_blocks, bn, out_dtype):
    B = x.shape[0]
    body = functools.partial(_edge_body, out_dtype=out_dtype)
    return pl.pallas_call(
        body,
        grid=(n_blocks,),
        in_specs=[
            pl.BlockSpec((B, k_size), lambda n: (0, 0)),
            pl.BlockSpec((bn, k_size), lambda n, _s=n_block_start: (n + _s, 0)),
        ],
        out_specs=pl.BlockSpec((B, bn), lambda n: (0, n)),
        out_shape=jax.ShapeDtypeStruct((B, n_blocks * bn), out_dtype),
        compiler_params=_CP,
    )(x, W)


def _mid_body(h1_ref, w_ref, o_ref, h_scr, *, bn):
    s = pl.program_id(0)
    n = pl.program_id(1)

    @pl.when(jnp.logical_and(s == 0, n == 0))
    def _():
        h_scr[0] = h1_ref[...]

    src = lax.rem(s, 2)
    dst = 1 - src
    acc = lax.dot_general(
        h_scr[src],
        w_ref[pl.ds(n * bn, bn), :],
        dimension_numbers=_NT,
        preferred_element_type=jnp.float32,
    )
    tile = jax.nn.sigmoid(acc).astype(jnp.bfloat16)
    h_scr[dst, :, pl.ds(n * bn, bn)] = tile
    o_ref[...] = tile


def _mid_steps(h1, Wb):
    B, H = h1.shape
    n_blocks = H // BN_
    body = functools.partial(_mid_body, bn=BN_)
    return pl.pallas_call(
        body,
        grid=(N_MID_STEPS_, n_blocks),
        in_specs=[
            pl.BlockSpec((B, H), lambda s, n: (0, 0)),
            pl.BlockSpec((H, H), lambda s, n: (0, 0)),
        ],
        out_specs=pl.BlockSpec((B, BN_), lambda s, n: (0, n)),
        out_shape=jax.ShapeDtypeStruct((B, H), jnp.bfloat16),
        scratch_shapes=[pltpu.VMEM((2, B, H), jnp.bfloat16)],
        compiler_params=pltpu.CompilerParams(
            dimension_semantics=("arbitrary", "arbitrary"),
            vmem_limit_bytes=110 * 1024 * 1024,
        ),
    )(h1, Wb)


def kernel(inp, W):
    B, n_inputs = inp.shape
    H = W.shape[0]
    Wb = W.astype(jnp.bfloat16)
    inp_bf = inp.astype(jnp.bfloat16)
    h1 = _edge_step(inp_bf, Wb[:, :n_inputs], k_size=n_inputs,
                    n_block_start=0, n_blocks=H // BN_, bn=BN_,
                    out_dtype=jnp.bfloat16)
    h7 = _mid_steps(h1, Wb)
    out = _edge_step(h7, Wb, k_size=H,
                     n_block_start=(H - N_OUT_) // N_OUT_, n_blocks=1,
                     bn=N_OUT_, out_dtype=jnp.float32)
    return out


# speedup vs baseline: 1.0953x; 1.0408x over previous
import functools

import jax
import jax.numpy as jnp
from jax import lax
from jax.experimental import pallas as pl
from jax.experimental.pallas import tpu as pltpu

N_OUT_ = 128
N_MID_STEPS_ = 6
BN_ = 512

_CP = pltpu.CompilerParams(vmem_limit_bytes=110 * 1024 * 1024)
_NT = (((1,), (1,)), ((), ()))


def _edge_body(x_ref, w_ref, o_ref, *, out_dtype):
    acc = lax.dot_general(x_ref[...], w_ref[...], dimension_numbers=_NT,
                          preferred_element_type=jnp.float32)
    o_ref[...] = jax.nn.sigmoid(acc).astype(out_dtype)


def _edge_step(x, W, *, k_size, n_block_start, n_blocks, bn, out_dtype):
    B = x.shape[0]
    body = functools.partial(_edge_body, out_dtype=out_dtype)
    return pl.pallas_call(
        body,
        grid=(n_blocks,),
        in_specs=[
            pl.BlockSpec((B, k_size), lambda n: (0, 0)),
            pl.BlockSpec((bn, k_size), lambda n, _s=n_block_start: (n + _s, 0)),
        ],
        out_specs=pl.BlockSpec((B, bn), lambda n: (0, n)),
        out_shape=jax.ShapeDtypeStruct((B, n_blocks * bn), out_dtype),
        compiler_params=_CP,
    )(x, W)


def _mid_body(h1_ref, w_ref, o_ref, h_scr, *, bn):
    s = pl.program_id(0)
    n = pl.program_id(1)

    @pl.when(jnp.logical_and(s == 0, n == 0))
    def _():
        h_scr[0] = h1_ref[...]

    src = lax.rem(s, 2)
    dst = 1 - src
    acc = lax.dot_general(
        h_scr[src],
        w_ref[pl.ds(n * bn, bn), :],
        dimension_numbers=_NT,
        preferred_element_type=jnp.float32,
    )
    tile = jax.nn.sigmoid(acc).astype(jnp.bfloat16)
    h_scr[dst, :, pl.ds(n * bn, bn)] = tile
    o_ref[...] = tile


def _mid_steps(h1, Wb):
    B, H = h1.shape
    n_blocks = H // BN_
    body = functools.partial(_mid_body, bn=BN_)
    return pl.pallas_call(
        body,
        grid=(N_MID_STEPS_, n_blocks),
        in_specs=[
            pl.BlockSpec((B, H), lambda s, n: (0, 0)),
            pl.BlockSpec((H, H), lambda s, n: (0, 0)),
        ],
        out_specs=pl.BlockSpec((B, BN_), lambda s, n: (0, n)),
        out_shape=jax.ShapeDtypeStruct((B, H), jnp.bfloat16),
        scratch_shapes=[pltpu.VMEM((2, B, H), jnp.bfloat16)],
        compiler_params=pltpu.CompilerParams(
            dimension_semantics=("arbitrary", "arbitrary"),
            vmem_limit_bytes=110 * 1024 * 1024,
        ),
    )(h1, Wb)


def kernel(inp, W):
    B, n_inputs = inp.shape
    H = W.shape[0]
    Wb = W.astype(jnp.bfloat16)
    inp_bf = inp.astype(jnp.bfloat16)
    h1 = _edge_step(inp_bf, Wb[:, :n_inputs], k_size=n_inputs,
                    n_block_start=0, n_blocks=H // BN_, bn=BN_,
                    out_dtype=jnp.bfloat16)
    h7 = _mid_steps(h1, Wb)
    out = _edge_step(h7, Wb, k_size=H,
                     n_block_start=(H - N_OUT_) // N_OUT_, n_blocks=1,
                     bn=N_OUT_, out_dtype=jnp.float32)
    return out


# speedup vs baseline: 1.1618x; 1.0607x over previous
import functools

import jax
import jax.numpy as jnp
from jax import lax
from jax.experimental import pallas as pl
from jax.experimental.pallas import tpu as pltpu

N_OUT_ = 128
N_STEPS_ = 8
BN_ = 512

_NT = (((1,), (1,)), ((), ()))


def _body(inp_ref, w_ref, o_ref, h_scr, *, bn, n_blocks, n_in, n_out):
    t = pl.program_id(0)
    t_last = (N_STEPS_ - 1) * n_blocks

    @pl.when(t < n_blocks)
    def _():
        n = t
        acc = lax.dot_general(
            inp_ref[...], w_ref[pl.ds(n * bn, bn), :n_in],
            dimension_numbers=_NT, preferred_element_type=jnp.float32)
        h_scr[1, :, pl.ds(n * bn, bn)] = jax.nn.sigmoid(acc).astype(jnp.bfloat16)

    @pl.when(jnp.logical_and(t >= n_blocks, t < t_last))
    def _():
        s = t // n_blocks
        n = lax.rem(t, n_blocks)
        src = lax.rem(s, 2)
        acc = lax.dot_general(
            h_scr[src], w_ref[pl.ds(n * bn, bn), :],
            dimension_numbers=_NT, preferred_element_type=jnp.float32)
        h_scr[1 - src, :, pl.ds(n * bn, bn)] = (
            jax.nn.sigmoid(acc).astype(jnp.bfloat16))

    @pl.when(t == t_last)
    def _():
        H = w_ref.shape[0]
        acc = lax.dot_general(
            h_scr[1], w_ref[pl.ds(H - n_out, n_out), :],
            dimension_numbers=_NT, preferred_element_type=jnp.float32)
        o_ref[...] = jax.nn.sigmoid(acc)


def kernel(inp, W):
    B, n_inputs = inp.shape
    H = W.shape[0]
    n_blocks = H // BN_
    n_iters = (N_STEPS_ - 1) * n_blocks + 1
    body = functools.partial(_body, bn=BN_, n_blocks=n_blocks,
                             n_in=n_inputs, n_out=N_OUT_)
    return pl.pallas_call(
        body,
        grid=(n_iters,),
        in_specs=[
            pl.BlockSpec((B, n_inputs), lambda t: (0, 0)),
            pl.BlockSpec((H, H), lambda t: (0, 0)),
        ],
        out_specs=pl.BlockSpec((B, N_OUT_), lambda t: (0, 0)),
        out_shape=jax.ShapeDtypeStruct((B, N_OUT_), jnp.float32),
        scratch_shapes=[pltpu.VMEM((2, B, H), jnp.bfloat16)],
        compiler_params=pltpu.CompilerParams(
            dimension_semantics=("arbitrary",),
            vmem_limit_bytes=110 * 1024 * 1024,
        ),
    )(inp.astype(jnp.bfloat16), W.astype(jnp.bfloat16))


# speedup vs baseline: 1.2118x; 1.0431x over previous
import functools

import jax
import jax.numpy as jnp
from jax import lax
from jax.experimental import pallas as pl
from jax.experimental.pallas import tpu as pltpu

N_OUT_ = 128
N_STEPS_ = 8
BN_ = 1024

_NT = (((1,), (1,)), ((), ()))


def _body(inp_ref, w_ref, o_ref, h_scr, *, bn, n_blocks, n_in, n_out):
    t = pl.program_id(0)
    t_last = (N_STEPS_ - 1) * n_blocks

    @pl.when(t < n_blocks)
    def _():
        n = t
        acc = lax.dot_general(
            inp_ref[...], w_ref[pl.ds(n * bn, bn), :n_in],
            dimension_numbers=_NT, preferred_element_type=jnp.float32)
        h_scr[1, :, pl.ds(n * bn, bn)] = jax.nn.sigmoid(acc).astype(jnp.bfloat16)

    @pl.when(jnp.logical_and(t >= n_blocks, t < t_last))
    def _():
        s = t // n_blocks
        n = lax.rem(t, n_blocks)
        src = lax.rem(s, 2)
        acc = lax.dot_general(
            h_scr[src], w_ref[pl.ds(n * bn, bn), :],
            dimension_numbers=_NT, preferred_element_type=jnp.float32)
        h_scr[1 - src, :, pl.ds(n * bn, bn)] = (
            jax.nn.sigmoid(acc).astype(jnp.bfloat16))

    @pl.when(t == t_last)
    def _():
        H = w_ref.shape[0]
        acc = lax.dot_general(
            h_scr[1], w_ref[pl.ds(H - n_out, n_out), :],
            dimension_numbers=_NT, preferred_element_type=jnp.float32)
        o_ref[...] = jax.nn.sigmoid(acc)


def kernel(inp, W):
    B, n_inputs = inp.shape
    H = W.shape[0]
    n_blocks = H // BN_
    n_iters = (N_STEPS_ - 1) * n_blocks + 1
    body = functools.partial(_body, bn=BN_, n_blocks=n_blocks,
                             n_in=n_inputs, n_out=N_OUT_)
    return pl.pallas_call(
        body,
        grid=(n_iters,),
        in_specs=[
            pl.BlockSpec((B, n_inputs), lambda t: (0, 0)),
            pl.BlockSpec((H, H), lambda t: (0, 0)),
        ],
        out_specs=pl.BlockSpec((B, N_OUT_), lambda t: (0, 0)),
        out_shape=jax.ShapeDtypeStruct((B, N_OUT_), jnp.float32),
        scratch_shapes=[pltpu.VMEM((2, B, H), jnp.bfloat16)],
        compiler_params=pltpu.CompilerParams(
            dimension_semantics=("arbitrary",),
            vmem_limit_bytes=110 * 1024 * 1024,
        ),
    )(inp.astype(jnp.bfloat16), W.astype(jnp.bfloat16))


# speedup vs baseline: 1.2310x; 1.0158x over previous
import functools

import jax
import jax.numpy as jnp
from jax import lax
from jax.experimental import pallas as pl
from jax.experimental.pallas import tpu as pltpu

N_OUT_ = 128
N_STEPS_ = 8
BN_ = 2048

_NT = (((1,), (1,)), ((), ()))


def _body(inp_ref, w_ref, o_ref, h_scr, *, bn, n_blocks, n_in, n_out):
    t = pl.program_id(0)
    t_last = (N_STEPS_ - 1) * n_blocks

    @pl.when(t < n_blocks)
    def _():
        n = t
        acc = lax.dot_general(
            inp_ref[...], w_ref[pl.ds(n * bn, bn), :n_in],
            dimension_numbers=_NT, preferred_element_type=jnp.float32)
        h_scr[1, :, pl.ds(n * bn, bn)] = jax.nn.sigmoid(acc).astype(jnp.bfloat16)

    @pl.when(jnp.logical_and(t >= n_blocks, t < t_last))
    def _():
        s = t // n_blocks
        n = lax.rem(t, n_blocks)
        src = lax.rem(s, 2)
        acc = lax.dot_general(
            h_scr[src], w_ref[pl.ds(n * bn, bn), :],
            dimension_numbers=_NT, preferred_element_type=jnp.float32)
        h_scr[1 - src, :, pl.ds(n * bn, bn)] = (
            jax.nn.sigmoid(acc).astype(jnp.bfloat16))

    @pl.when(t == t_last)
    def _():
        H = w_ref.shape[0]
        acc = lax.dot_general(
            h_scr[1], w_ref[pl.ds(H - n_out, n_out), :],
            dimension_numbers=_NT, preferred_element_type=jnp.float32)
        o_ref[...] = jax.nn.sigmoid(acc)


def kernel(inp, W):
    B, n_inputs = inp.shape
    H = W.shape[0]
    n_blocks = H // BN_
    n_iters = (N_STEPS_ - 1) * n_blocks + 1
    body = functools.partial(_body, bn=BN_, n_blocks=n_blocks,
                             n_in=n_inputs, n_out=N_OUT_)
    return pl.pallas_call(
        body,
        grid=(n_iters,),
        in_specs=[
            pl.BlockSpec((B, n_inputs), lambda t: (0, 0)),
            pl.BlockSpec((H, H), lambda t: (0, 0)),
        ],
        out_specs=pl.BlockSpec((B, N_OUT_), lambda t: (0, 0)),
        out_shape=jax.ShapeDtypeStruct((B, N_OUT_), jnp.float32),
        scratch_shapes=[pltpu.VMEM((2, B, H), jnp.bfloat16)],
        compiler_params=pltpu.CompilerParams(
            dimension_semantics=("arbitrary",),
            vmem_limit_bytes=110 * 1024 * 1024,
        ),
    )(inp.astype(jnp.bfloat16), W.astype(jnp.bfloat16))


# speedup vs baseline: 1.2420x; 1.0089x over previous
import functools

import jax
import jax.numpy as jnp
from jax import lax
from jax.experimental import pallas as pl
from jax.experimental.pallas import tpu as pltpu

N_OUT_ = 128
N_STEPS_ = 8

_NT = (((1,), (1,)), ((), ()))


def _body(inp_ref, w_ref, o_ref, h_scr, *, n_in, n_out):
    t = pl.program_id(0)
    H = w_ref.shape[0]

    @pl.when(t == 0)
    def _():
        acc = lax.dot_general(
            inp_ref[...], w_ref[:, :n_in],
            dimension_numbers=_NT, preferred_element_type=jnp.float32)
        h_scr[...] = jax.nn.sigmoid(acc).astype(jnp.bfloat16)

    @pl.when(jnp.logical_and(t >= 1, t < N_STEPS_ - 1))
    def _():
        acc = lax.dot_general(
            h_scr[...], w_ref[...],
            dimension_numbers=_NT, preferred_element_type=jnp.float32)
        h_scr[...] = jax.nn.sigmoid(acc).astype(jnp.bfloat16)

    @pl.when(t == N_STEPS_ - 1)
    def _():
        acc = lax.dot_general(
            h_scr[...], w_ref[pl.ds(H - n_out, n_out), :],
            dimension_numbers=_NT, preferred_element_type=jnp.float32)
        o_ref[...] = jax.nn.sigmoid(acc)


def kernel(inp, W):
    B, n_inputs = inp.shape
    H = W.shape[0]
    body = functools.partial(_body, n_in=n_inputs, n_out=N_OUT_)
    return pl.pallas_call(
        body,
        grid=(N_STEPS_,),
        in_specs=[
            pl.BlockSpec((B, n_inputs), lambda t: (0, 0)),
            pl.BlockSpec((H, H), lambda t: (0, 0)),
        ],
        out_specs=pl.BlockSpec((B, N_OUT_), lambda t: (0, 0)),
        out_shape=jax.ShapeDtypeStruct((B, N_OUT_), jnp.float32),
        scratch_shapes=[pltpu.VMEM((B, H), jnp.bfloat16)],
        compiler_params=pltpu.CompilerParams(
            dimension_semantics=("arbitrary",),
            vmem_limit_bytes=110 * 1024 * 1024,
        ),
    )(inp.astype(jnp.bfloat16), W.astype(jnp.bfloat16))


# speedup vs baseline: 1.3519x; 1.0885x over previous
import functools

import jax
import jax.numpy as jnp
from jax import lax
from jax.experimental import pallas as pl
from jax.experimental.pallas import tpu as pltpu

N_OUT_ = 128
N_STEPS_ = 8
BW_ = 256
BM_ = 512

_NT = (((1,), (1,)), ((), ()))


def _body(inp_ref, w_ref, o_ref, wbf_scr, h_scr, *, n_in, n_out, n_wblk,
          n_mblk):
    t = pl.program_id(0)
    H = wbf_scr.shape[0]
    t_mid0 = n_wblk
    t_last = n_wblk + (N_STEPS_ - 2) * n_mblk

    @pl.when(t < t_mid0)
    def _():
        wblk = w_ref[...].astype(jnp.bfloat16)
        wbf_scr[pl.ds(t * BW_, BW_), :] = wblk
        acc = lax.dot_general(
            inp_ref[...], wblk[:, :n_in],
            dimension_numbers=_NT, preferred_element_type=jnp.float32)
        h_scr[:, pl.ds(t * BW_, BW_)] = jax.nn.sigmoid(acc).astype(jnp.bfloat16)

    @pl.when(jnp.logical_and(t >= t_mid0, t < t_last))
    def _():
        m = lax.rem(t - t_mid0, n_mblk)
        rows = pl.ds(m * BM_, BM_)
        acc = lax.dot_general(
            h_scr[rows, :], wbf_scr[...],
            dimension_numbers=_NT, preferred_element_type=jnp.float32)
        h_scr[rows, :] = jax.nn.sigmoid(acc).astype(jnp.bfloat16)

    @pl.when(t == t_last)
    def _():
        acc = lax.dot_general(
            h_scr[...], wbf_scr[pl.ds(H - n_out, n_out), :],
            dimension_numbers=_NT, preferred_element_type=jnp.float32)
        o_ref[...] = jax.nn.sigmoid(acc)


def kernel(inp, W):
    B, n_inputs = inp.shape
    H = W.shape[0]
    n_wblk = H // BW_
    n_mblk = B // BM_
    n_iters = n_wblk + (N_STEPS_ - 2) * n_mblk + 1
    body = functools.partial(_body, n_in=n_inputs, n_out=N_OUT_,
                             n_wblk=n_wblk, n_mblk=n_mblk)
    last_w = n_wblk - 1
    return pl.pallas_call(
        body,
        grid=(n_iters,),
        in_specs=[
            pl.BlockSpec((B, n_inputs), lambda t: (0, 0)),
            pl.BlockSpec((BW_, H), lambda t: (jnp.minimum(t, last_w), 0)),
        ],
        out_specs=pl.BlockSpec((B, N_OUT_), lambda t: (0, 0)),
        out_shape=jax.ShapeDtypeStruct((B, N_OUT_), jnp.float32),
        scratch_shapes=[
            pltpu.VMEM((H, H), jnp.bfloat16),
            pltpu.VMEM((B, H), jnp.bfloat16),
        ],
        compiler_params=pltpu.CompilerParams(
            dimension_semantics=("arbitrary",),
            vmem_limit_bytes=110 * 1024 * 1024,
        ),
    )(inp.astype(jnp.bfloat16), W)


# speedup vs baseline: 1.3749x; 1.0170x over previous
import functools

import jax
import jax.numpy as jnp
from jax import lax
from jax.experimental import pallas as pl
from jax.experimental.pallas import tpu as pltpu

N_OUT_ = 128
N_STEPS_ = 8
BW_ = 256
BM_ = 256

_NT = (((1,), (1,)), ((), ()))


def _body(inp_ref, w_ref, o_ref, wbf_scr, h_scr, *, n_in, n_out, n_wblk,
          n_mblk):
    t = pl.program_id(0)
    H = wbf_scr.shape[0]
    t_mid0 = n_wblk
    t_last = n_wblk + (N_STEPS_ - 2) * n_mblk

    @pl.when(t < t_mid0)
    def _():
        wblk = w_ref[...].astype(jnp.bfloat16)
        wbf_scr[pl.ds(t * BW_, BW_), :] = wblk
        acc = lax.dot_general(
            inp_ref[...], wblk[:, :n_in],
            dimension_numbers=_NT, preferred_element_type=jnp.float32)
        h_scr[:, pl.ds(t * BW_, BW_)] = jax.nn.sigmoid(acc).astype(jnp.bfloat16)

    @pl.when(jnp.logical_and(t >= t_mid0, t < t_last))
    def _():
        m = lax.rem(t - t_mid0, n_mblk)
        rows = pl.ds(m * BM_, BM_)
        acc = lax.dot_general(
            h_scr[rows, :], wbf_scr[...],
            dimension_numbers=_NT, preferred_element_type=jnp.float32)
        h_scr[rows, :] = jax.nn.sigmoid(acc).astype(jnp.bfloat16)

    @pl.when(t == t_last)
    def _():
        acc = lax.dot_general(
            h_scr[...], wbf_scr[pl.ds(H - n_out, n_out), :],
            dimension_numbers=_NT, preferred_element_type=jnp.float32)
        o_ref[...] = jax.nn.sigmoid(acc)


def kernel(inp, W):
    B, n_inputs = inp.shape
    H = W.shape[0]
    n_wblk = H // BW_
    n_mblk = B // BM_
    n_iters = n_wblk + (N_STEPS_ - 2) * n_mblk + 1
    body = functools.partial(_body, n_in=n_inputs, n_out=N_OUT_,
                             n_wblk=n_wblk, n_mblk=n_mblk)
    last_w = n_wblk - 1
    return pl.pallas_call(
        body,
        grid=(n_iters,),
        in_specs=[
            pl.BlockSpec((B, n_inputs), lambda t: (0, 0)),
            pl.BlockSpec((BW_, H), lambda t: (jnp.minimum(t, last_w), 0)),
        ],
        out_specs=pl.BlockSpec((B, N_OUT_), lambda t: (0, 0)),
        out_shape=jax.ShapeDtypeStruct((B, N_OUT_), jnp.float32),
        scratch_shapes=[
            pltpu.VMEM((H, H), jnp.bfloat16),
            pltpu.VMEM((B, H), jnp.bfloat16),
        ],
        compiler_params=pltpu.CompilerParams(
            dimension_semantics=("arbitrary",),
            vmem_limit_bytes=110 * 1024 * 1024,
        ),
    )(inp.astype(jnp.bfloat16), W)
